# baseline (speedup 1.0000x reference)
import functools
import math

import jax
import jax.numpy as jnp
from jax import lax
from jax.experimental import pallas as pl
from jax.experimental.pallas import tpu as pltpu
from jax.experimental.pallas import tpu_sc as plsc

L = 200
B = 4096
DIM = 64
N = L * B
SCALE = math.sqrt(DIM)

_info = plsc.get_sparse_core_info()
NC, NS = _info.num_cores, _info.num_subcores
NW = NC * NS
CHUNK = B // NW


def _sc_body(idx_hbm, w_hbm, pe_hbm, out_hbm, pe_v, idx_v, rows_v, sem):
    wid = lax.axis_index("s") * NC + lax.axis_index("c")
    pltpu.sync_copy(pe_hbm, pe_v)

    def step(l, _):
        base = pl.multiple_of(l * B + wid * CHUNK, 8)
        pltpu.sync_copy(idx_hbm.at[pl.ds(base, CHUNK)], idx_v)
        pltpu.async_copy(w_hbm.at[idx_v], rows_v, sem).wait()
        pe_regs = [pe_v[l, pl.ds(16 * j, 16)] for j in range(DIM // 16)]

        def fma(r, _):
            for j in range(DIM // 16):
                sl = pl.ds(16 * j, 16)
                rows_v[r, sl] = rows_v[r, sl] * SCALE + pe_regs[j]
            return 0

        lax.fori_loop(0, CHUNK, fma, 0)
        pltpu.sync_copy(rows_v, out_hbm.at[pl.ds(base, CHUNK)])
        return 0

    lax.fori_loop(0, L, step, 0)


@jax.jit
def _embed(idx, W, pe_s):
    mesh = plsc.VectorSubcoreMesh(core_axis_name="c", subcore_axis_name="s")
    f = pl.kernel(
        _sc_body,
        out_type=jax.ShapeDtypeStruct((N, DIM), jnp.float32),
        mesh=mesh,
        scratch_types=[
            pltpu.VMEM((L, DIM), jnp.float32),
            pltpu.VMEM((CHUNK,), jnp.int32),
            pltpu.VMEM((CHUNK, DIM), jnp.float32),
            pltpu.SemaphoreType.DMA,
        ],
        compiler_params=pltpu.CompilerParams(use_tc_tiling_on_sc=False),
    )
    return f(idx, W, pe_s)


def kernel(source, W, pe):
    idx = source.reshape(N)
    pe_s = pe[:L, 0, :]
    out = _embed(idx, W, pe_s)
    return out.reshape(L, B, DIM)


# speedup vs baseline: 1.3073x; 1.3073x over previous
import math

import jax
import jax.numpy as jnp
from jax import lax
from jax.experimental import pallas as pl
from jax.experimental.pallas import tpu as pltpu
from jax.experimental.pallas import tpu_sc as plsc

L = 200
B = 4096
DIM = 64
N = L * B
SCALE = math.sqrt(DIM)

_info = plsc.get_sparse_core_info()
NC, NS = _info.num_cores, _info.num_subcores
NW = NC * NS
CH = B // NW
NBUF = 4
ROUNDS = L // NBUF


def _sc_body(idx_hbm, w_hbm, pe_hbm, out_hbm, pe_v, idx_v, rin, rout, *sems):
    gsem = sems[:NBUF]
    ssem = sems[NBUF:]
    wid = lax.axis_index("s") * NC + lax.axis_index("c")
    col = wid * CH
    pltpu.sync_copy(pe_hbm, pe_v)
    pltpu.sync_copy(idx_hbm.at[:, pl.ds(col, CH)], idx_v)

    def out_slice(l):
        base = pl.multiple_of(l * B + col, 8)
        return out_hbm.at[pl.ds(base, CH)]

    def fire_gather(l, b):
        pltpu.async_copy(w_hbm.at[idx_v.at[l]], rin.at[b], gsem[b])

    def wait_gather(l, b):
        pltpu.make_async_copy(w_hbm.at[idx_v.at[l]], rin.at[b], gsem[b]).wait()

    def fire_store(l, b):
        pltpu.async_copy(rout.at[b], out_slice(l), ssem[b])

    def wait_store(l, b):
        pltpu.make_async_copy(rout.at[b], out_slice(l), ssem[b]).wait()

    def fma(l, b):
        pe_regs = [pe_v[l, pl.ds(16 * j, 16)] for j in range(DIM // 16)]

        @plsc.parallel_loop(0, CH, step=1, unroll=4)
        def _(r):
            for j in range(DIM // 16):
                sl = pl.ds(16 * j, 16)
                rout[b, r, sl] = rin[b, r, sl] * SCALE + pe_regs[j]

    def step(l, b, first, fire_next):
        wait_gather(l, b)
        if not first:
            wait_store(l - NBUF, b)
        fma(l, b)
        fire_store(l, b)
        if fire_next:
            fire_gather(l + NBUF, b)

    for b in range(NBUF):
        fire_gather(b, b)
    for b in range(NBUF):
        step(b, b, first=True, fire_next=True)

    def round_body(mc, _):
        for b in range(NBUF):
            step(mc * NBUF + b, b, first=False, fire_next=True)
        return 0

    lax.fori_loop(1, ROUNDS - 1, round_body, 0)

    last = (ROUNDS - 1) * NBUF
    for b in range(NBUF):
        step(last + b, b, first=False, fire_next=False)
    for b in range(NBUF):
        wait_store(last + b, b)


@jax.jit
def _embed(idx, W, pe_s):
    mesh = plsc.VectorSubcoreMesh(core_axis_name="c", subcore_axis_name="s")
    f = pl.kernel(
        _sc_body,
        out_type=jax.ShapeDtypeStruct((N, DIM), jnp.float32),
        mesh=mesh,
        scratch_types=[
            pltpu.VMEM((L, DIM), jnp.float32),
            pltpu.VMEM((L, CH), jnp.int32),
            pltpu.VMEM((NBUF, CH, DIM), jnp.float32),
            pltpu.VMEM((NBUF, CH, DIM), jnp.float32),
        ]
        + [pltpu.SemaphoreType.DMA] * (2 * NBUF),
        compiler_params=pltpu.CompilerParams(use_tc_tiling_on_sc=False),
    )
    return f(idx, W, pe_s)


def kernel(source, W, pe):
    idx = source.reshape(L, B)
    pe_s = pe[:L, 0, :]
    out = _embed(idx, W, pe_s)
    return out.reshape(L, B, DIM)
